# baseline (speedup 1.0000x reference)
import functools

import jax
import jax.numpy as jnp
from jax import lax
from jax.experimental import pallas as pl
from jax.experimental.pallas import tpu as pltpu
from jax.experimental.pallas import tpu_sc as plsc

B, L, D = 4, 4096, 1024
NC, NS = 2, 16
NW = NC * NS
RPW = L // NW
CH = 8
CHE = CH * D
NCH = RPW // CH
NSTEP = NCH * B
NX = 6
AHEAD = NX - 1
NP = 2
LANES = 16
UNROLL = 8


def _sc_body(x_hbm, pos_hbm, out_hbm, *refs):
    xbufs = list(refs[:NX])
    pbufs = list(refs[NX:NX + NP])
    sem_l = list(refs[NX + NP:2 * NX + NP])
    sem_s = list(refs[2 * NX + NP:3 * NX + NP])
    sem_p = list(refs[3 * NX + NP:])

    wid = lax.axis_index("s") * NC + lax.axis_index("c")
    pel0 = wid * (RPW * D)

    def xload(k):
        c, b = k // B, k % B
        off = b * (L * D) + pel0 + c * CHE
        return pltpu.make_async_copy(
            x_hbm.at[pl.ds(off, CHE)], xbufs[k % NX], sem_l[k % NX])

    def xstore(k):
        c, b = k // B, k % B
        off = b * (L * D) + pel0 + c * CHE
        return pltpu.make_async_copy(
            xbufs[k % NX], out_hbm.at[pl.ds(off, CHE)], sem_s[k % NX])

    def pload(c):
        return pltpu.make_async_copy(
            pos_hbm.at[pl.ds(pel0 + c * CHE, CHE)], pbufs[c % NP], sem_p[c % NP])

    pload(0).start()
    pload(1).start()
    for t in range(NSTEP + AHEAD):
        if t < NSTEP:
            if t >= NX:
                xstore(t - NX).wait()
            xload(t).start()
        k = t - AHEAD
        if 0 <= k < NSTEP:
            c, b = k // B, k % B
            xload(k).wait()
            if b == 0:
                pload(c).wait()
            xb = xbufs[k % NX]
            pb = pbufs[c % NP]

            def add_body(i, _):
                base = i * (LANES * UNROLL)
                for u in range(UNROLL):
                    sl = pl.ds(base + u * LANES, LANES)
                    xb[sl] = xb[sl] + pb[sl]
                return 0

            lax.fori_loop(0, CHE // (LANES * UNROLL), add_body, 0)
            xstore(k).start()
            if b == B - 1 and c + 2 < NCH:
                pload(c + 2).start()
    for k in range(NSTEP - NX, NSTEP):
        xstore(k).wait()


_sc_add = functools.partial(
    pl.kernel,
    mesh=plsc.VectorSubcoreMesh(core_axis_name="c", subcore_axis_name="s"),
    out_type=jax.ShapeDtypeStruct((B * L * D,), jnp.float32),
    scratch_types=(
        [pltpu.VMEM((CHE,), jnp.float32) for _ in range(NX + NP)]
        + [pltpu.SemaphoreType.DMA] * (2 * NX + NP)
    ),
)(_sc_body)


def kernel(x, pos_table):
    out = _sc_add(x.reshape(-1), pos_table.reshape(-1))
    return out.reshape(B, L, D)


# speedup vs baseline: 1.4500x; 1.4500x over previous
import functools

import jax
import jax.numpy as jnp
from jax import lax
from jax.experimental import pallas as pl
from jax.experimental.pallas import tpu as pltpu
from jax.experimental.pallas import tpu_sc as plsc

B, L, D = 4, 4096, 1024
NC, NS = 2, 16
NW = NC * NS

L_SC = 512
L_TC = L - L_SC


RPW = L_SC // NW
CH = 16
CHE = CH * D
NCH = RPW // CH
NSTEP = NCH * B
NX = 3
AHEAD = 2
NP = 2
LANES = 16
UNROLL = 8


def _sc_body(x_hbm, pos_hbm, out_hbm, *refs):
    xbufs = list(refs[:NX])
    pbufs = list(refs[NX:NX + NP])
    sem_l = list(refs[NX + NP:2 * NX + NP])
    sem_s = list(refs[2 * NX + NP:3 * NX + NP])
    sem_p = list(refs[3 * NX + NP:])

    wid = lax.axis_index("s") * NC + lax.axis_index("c")
    pel0 = (L_TC + wid * RPW) * D
    oel0 = wid * (RPW * D)

    def xload(k):
        c, b = k // B, k % B
        off = b * (L * D) + pel0 + c * CHE
        return pltpu.make_async_copy(
            x_hbm.at[pl.ds(off, CHE)], xbufs[k % NX], sem_l[k % NX])

    def xstore(k):
        c, b = k // B, k % B
        off = b * (L_SC * D) + oel0 + c * CHE
        return pltpu.make_async_copy(
            xbufs[k % NX], out_hbm.at[pl.ds(off, CHE)], sem_s[k % NX])

    def pload(c):
        return pltpu.make_async_copy(
            pos_hbm.at[pl.ds(pel0 + c * CHE, CHE)], pbufs[c % NP], sem_p[c % NP])

    pload(0).start()
    if NCH > 1:
        pload(1).start()
    for t in range(NSTEP + AHEAD):
        if t < NSTEP:
            if t >= NX:
                xstore(t - NX).wait()
            xload(t).start()
        k = t - AHEAD
        if 0 <= k < NSTEP:
            c, b = k // B, k % B
            xload(k).wait()
            if b == 0:
                pload(c).wait()
            xb = xbufs[k % NX]
            pb = pbufs[c % NP]

            def add_body(i, _):
                base = i * (LANES * UNROLL)
                for u in range(UNROLL):
                    sl = pl.ds(base + u * LANES, LANES)
                    xb[sl] = xb[sl] + pb[sl]
                return 0

            lax.fori_loop(0, CHE // (LANES * UNROLL), add_body, 0)
            xstore(k).start()
            if b == B - 1 and c + 2 < NCH:
                pload(c + 2).start()
    for k in range(max(0, NSTEP - NX), NSTEP):
        xstore(k).wait()


_sc_tail = functools.partial(
    pl.kernel,
    mesh=plsc.VectorSubcoreMesh(core_axis_name="c", subcore_axis_name="s"),
    out_type=jax.ShapeDtypeStruct((B * L_SC * D,), jnp.float32),
    scratch_types=(
        [pltpu.VMEM((CHE,), jnp.float32) for _ in range(NX + NP)]
        + [pltpu.SemaphoreType.DMA] * (2 * NX + NP)
    ),
)(_sc_body)


LB = 512


def _tc_add(x_ref, pos_ref, out_ref):
    out_ref[...] = x_ref[...] + pos_ref[...]


def _tc_head(x, pos_table):
    grid = (L_TC // LB, B)
    return pl.pallas_call(
        _tc_add,
        grid=grid,
        in_specs=[
            pl.BlockSpec((1, LB, D), lambda l, b: (b, l, 0)),
            pl.BlockSpec((LB, D), lambda l, b: (l, 0)),
        ],
        out_specs=pl.BlockSpec((1, LB, D), lambda l, b: (b, l, 0)),
        out_shape=jax.ShapeDtypeStruct((B, L, D), x.dtype),
        compiler_params=pltpu.CompilerParams(
            dimension_semantics=("arbitrary", "arbitrary"),
        ),
    )(x, pos_table)


def kernel(x, pos_table):
    sc_part = _sc_tail(x.reshape(-1), pos_table.reshape(-1))
    tc_out = _tc_head(x, pos_table)
    return lax.dynamic_update_slice(
        tc_out, sc_part.reshape(B, L_SC, D), (0, L_TC, 0))


# speedup vs baseline: 4.7719x; 3.2910x over previous
import jax
import jax.numpy as jnp
from jax.experimental import pallas as pl
from jax.experimental.pallas import tpu as pltpu

LB = 2048


def _add_kernel(x_ref, pos_ref, out_ref):
    out_ref[...] = x_ref[...] + pos_ref[...]


def kernel(x, pos_table):
    B, L, D = x.shape
    grid = (L // LB, B)
    return pl.pallas_call(
        _add_kernel,
        grid=grid,
        in_specs=[
            pl.BlockSpec((1, LB, D), lambda l, b: (b, l, 0)),
            pl.BlockSpec((LB, D), lambda l, b: (l, 0)),
        ],
        out_specs=pl.BlockSpec((1, LB, D), lambda l, b: (b, l, 0)),
        out_shape=jax.ShapeDtypeStruct((B, L, D), x.dtype),
        compiler_params=pltpu.CompilerParams(
            dimension_semantics=("arbitrary", "arbitrary"),
        ),
    )(x, pos_table[:L])
